# baseline (speedup 1.0000x reference)
import functools

import jax
import jax.numpy as jnp
from jax import lax
from jax.experimental import pallas as pl
from jax.experimental.pallas import tpu as pltpu
from jax.experimental.pallas import tpu_sc as plsc

_B, _R, _C, _K, _S = 4, 8, 512, 256, 1024
_N = _B * _R * _C
_BN = 4096

_NC, _NS = 2, 16
_NW = _NC * _NS
_BPW = _N // _NW
_CH = 128
_NCH = _BPW // _CH


def _vq_block(v_ref, cb_ref, z_ref, err_ref):
    v = v_ref[...]
    cb = cb_ref[...]
    dots2 = jax.lax.dot_general(
        v * (-2.0), cb, (((1,), (1,)), ((), ())),
        preferred_element_type=jnp.float32)
    v2 = jnp.sum(v * v, axis=1, keepdims=True)
    c2 = jnp.sum(cb * cb, axis=1)
    diffs2 = (v2 + dots2) + c2[None, :]
    m = jnp.min(diffs2, axis=1, keepdims=True)
    sidxf = jax.lax.broadcasted_iota(
        jnp.int32, diffs2.shape, 1).astype(jnp.float32)
    zf = jnp.min(jnp.where(diffs2 == m, sidxf, float(_S)),
                 axis=1, keepdims=True)
    z_ref[...] = zf.astype(jnp.int32)
    partial = jnp.sum(jnp.maximum(m, 0.0))

    @pl.when(pl.program_id(0) == 0)
    def _init():
        err_ref[...] = jnp.zeros_like(err_ref)

    err_ref[...] += partial


def _vq_tc(vf, codebook):
    return pl.pallas_call(
        _vq_block,
        grid=(_N // _BN,),
        in_specs=[
            pl.BlockSpec((_BN, _K), lambda i: (i, 0)),
            pl.BlockSpec((_S, _K), lambda i: (0, 0)),
        ],
        out_specs=[
            pl.BlockSpec((_BN, 1), lambda i: (i, 0)),
            pl.BlockSpec((8, 128), lambda i: (0, 0)),
        ],
        out_shape=[
            jax.ShapeDtypeStruct((_N, 1), jnp.int32),
            jax.ShapeDtypeStruct((8, 128), jnp.float32),
        ],
    )(vf, codebook)


@functools.partial(
    pl.kernel,
    mesh=plsc.VectorSubcoreMesh(core_axis_name="c", subcore_axis_name="s"),
    out_type=jax.ShapeDtypeStruct((_NW, _NCH, _CH, _K), jnp.float32),
    scratch_types=[
        pltpu.VMEM((_NCH, _CH), jnp.int32),
        pltpu.VMEM((_CH, _K), jnp.float32),
        pltpu.VMEM((_CH, _K), jnp.float32),
        pltpu.SemaphoreType.DMA,
        pltpu.SemaphoreType.DMA,
    ],
)
def _sc_gather(z_hbm, cb_hbm, out_hbm, idx_v, rows0, rows1, sem0, sem1):
    wid = lax.axis_index("s") * _NC + lax.axis_index("c")
    pltpu.sync_copy(z_hbm.at[wid], idx_v)
    cp0 = pltpu.async_copy(cb_hbm.at[idx_v.at[0]], rows0, sem0)
    cp1 = pltpu.async_copy(cb_hbm.at[idx_v.at[1]], rows1, sem1)
    cp0.wait()
    pltpu.sync_copy(rows0, out_hbm.at[wid, 0])
    cp2 = pltpu.async_copy(cb_hbm.at[idx_v.at[2]], rows0, sem0)
    cp1.wait()
    pltpu.sync_copy(rows1, out_hbm.at[wid, 1])
    cp3 = pltpu.async_copy(cb_hbm.at[idx_v.at[3]], rows1, sem1)
    cp2.wait()
    pltpu.sync_copy(rows0, out_hbm.at[wid, 2])
    cp3.wait()
    pltpu.sync_copy(rows1, out_hbm.at[wid, 3])


def kernel(vecs, codebook):
    orig_dtype = vecs.dtype
    vf = vecs.astype(jnp.float32).reshape(_N, _K)
    z_col, err_acc = _vq_tc(vf, codebook)
    zg = z_col.reshape(_NW, _NCH, _CH)
    cz = _sc_gather(zg, codebook)
    z = z_col.reshape(_B, _R, _C)
    vecs_hat = cz.reshape(_B, _R, _C, _K).astype(orig_dtype)
    l = (err_acc[0, 0] / _N).astype(jnp.float32)
    return (vecs_hat, z, l, l)


# speedup vs baseline: 1.0002x; 1.0002x over previous
import functools

import jax
import jax.numpy as jnp
from jax import lax
from jax.experimental import pallas as pl
from jax.experimental.pallas import tpu as pltpu
from jax.experimental.pallas import tpu_sc as plsc

_B, _R, _C, _K, _S = 4, 8, 512, 256, 1024
_N = _B * _R * _C
_BN = 4096

_NC, _NS = 2, 16
_NW = _NC * _NS
_BPW = _N // _NW
_CH = 128
_NCH = _BPW // _CH


def _vq_block(v_ref, cb_ref, z_ref, err_ref):
    v = v_ref[...]
    cb = cb_ref[...]
    dots2 = jax.lax.dot_general(
        v * (-2.0), cb, (((1,), (1,)), ((), ())),
        preferred_element_type=jnp.float32)
    v2 = jnp.sum(v * v, axis=1, keepdims=True)
    c2 = jnp.sum(cb * cb, axis=1)
    diffs2 = (v2 + dots2) + c2[None, :]
    m = jnp.min(diffs2, axis=1, keepdims=True)
    sidxf = jax.lax.broadcasted_iota(
        jnp.int32, diffs2.shape, 1).astype(jnp.float32)
    zf = jnp.min(jnp.where(diffs2 == m, sidxf, float(_S)),
                 axis=1, keepdims=True)
    z_ref[...] = zf.astype(jnp.int32)
    partial = jnp.sum(jnp.maximum(m, 0.0))

    @pl.when(pl.program_id(0) == 0)
    def _init():
        err_ref[0, 0] = 0.0

    err_ref[0, 0] += partial


def _vq_tc(vf, codebook):
    return pl.pallas_call(
        _vq_block,
        grid=(_N // _BN,),
        in_specs=[
            pl.BlockSpec((_BN, _K), lambda i: (i, 0)),
            pl.BlockSpec((_S, _K), lambda i: (0, 0)),
        ],
        out_specs=[
            pl.BlockSpec((_BN, 1), lambda i: (i, 0)),
            pl.BlockSpec(memory_space=pltpu.SMEM),
        ],
        out_shape=[
            jax.ShapeDtypeStruct((_N, 1), jnp.int32),
            jax.ShapeDtypeStruct((1, 1), jnp.float32),
        ],
    )(vf, codebook)


@functools.partial(
    pl.kernel,
    mesh=plsc.VectorSubcoreMesh(core_axis_name="c", subcore_axis_name="s"),
    out_type=jax.ShapeDtypeStruct((_NW, _NCH, _CH, _K), jnp.float32),
    scratch_types=[
        pltpu.VMEM((_NCH, _CH), jnp.int32),
        pltpu.VMEM((_CH, _K), jnp.float32),
        pltpu.VMEM((_CH, _K), jnp.float32),
        pltpu.SemaphoreType.DMA,
        pltpu.SemaphoreType.DMA,
    ],
)
def _sc_gather(z_hbm, cb_hbm, out_hbm, idx_v, rows0, rows1, sem0, sem1):
    wid = lax.axis_index("s") * _NC + lax.axis_index("c")
    pltpu.sync_copy(z_hbm.at[wid], idx_v)
    cp0 = pltpu.async_copy(cb_hbm.at[idx_v.at[0]], rows0, sem0)
    cp1 = pltpu.async_copy(cb_hbm.at[idx_v.at[1]], rows1, sem1)
    cp0.wait()
    pltpu.sync_copy(rows0, out_hbm.at[wid, 0])
    cp2 = pltpu.async_copy(cb_hbm.at[idx_v.at[2]], rows0, sem0)
    cp1.wait()
    pltpu.sync_copy(rows1, out_hbm.at[wid, 1])
    cp3 = pltpu.async_copy(cb_hbm.at[idx_v.at[3]], rows1, sem1)
    cp2.wait()
    pltpu.sync_copy(rows0, out_hbm.at[wid, 2])
    cp3.wait()
    pltpu.sync_copy(rows1, out_hbm.at[wid, 3])


def kernel(vecs, codebook):
    orig_dtype = vecs.dtype
    vf = vecs.astype(jnp.float32).reshape(_N, _K)
    z_col, err_acc = _vq_tc(vf, codebook)
    zg = z_col.reshape(_NW, _NCH, _CH)
    cz = _sc_gather(zg, codebook)
    z = z_col.reshape(_B, _R, _C)
    vecs_hat = cz.reshape(_B, _R, _C, _K).astype(orig_dtype)
    l = (err_acc[0, 0] / _N).astype(jnp.float32)
    return (vecs_hat, z, l, l)
